# baseline (speedup 1.0000x reference)
import functools

import jax
import jax.numpy as jnp
from jax import lax
from jax.experimental import pallas as pl
from jax.experimental.pallas import tpu as pltpu


N_NODES = 10000
N_EDGES = 320000
LATENT = 128
BE = 1600


def _mlp_tc_body(x_ref, va_ref, vb_ref, vc_ref, w1_ref, b1_ref, w2_ref, b2_ref,
                 out_ref):
    x = x_ref[...]
    h = jnp.maximum(
        lax.dot_general(x, w1_ref[...], (((1,), (0,)), ((), ())),
                        precision=lax.Precision.HIGHEST,
                        preferred_element_type=jnp.float32) + b1_ref[...],
        0.0)
    o = lax.dot_general(h, w2_ref[...], (((1,), (0,)), ((), ())),
                        precision=lax.Precision.HIGHEST,
                        preferred_element_type=jnp.float32) + b2_ref[...]
    va = va_ref[...]
    vb = vb_ref[...]
    vc = vc_ref[...]
    fij = o[:, 0:1] * va + o[:, 1:2] * vb + o[:, 2:3] * vc
    aij = o[:, 3:4] * va + o[:, 4:5] * vb + o[:, 5:6] * vc
    dxij = o[:, 6:7] * va + o[:, 7:8] * vb + o[:, 8:9] * vc
    out_ref[...] = jnp.concatenate(
        [fij, aij, dxij, o[:, 9:10], jnp.zeros((x.shape[0], 6), jnp.float32)],
        axis=1)


def _decode_vals16(interaction_latent, vector_a, vector_b, vector_c,
                   i1_params, i2_params, fs_params, dx_params):
    w1 = jnp.concatenate(
        [i1_params[0], i2_params[0], dx_params[0], fs_params[0]], axis=1)
    b1 = jnp.concatenate(
        [i1_params[1], i2_params[1], dx_params[1], fs_params[1]], axis=0)
    z = jnp.zeros((LATENT, 16), jnp.float32)
    w2 = jnp.concatenate([
        z.at[:, 0:3].set(i1_params[2]),
        z.at[:, 3:6].set(i2_params[2]),
        z.at[:, 6:9].set(dx_params[2]),
        z.at[:, 9:10].set(fs_params[2]),
    ], axis=0)
    b2 = jnp.concatenate([
        i1_params[3], i2_params[3], dx_params[3], fs_params[3],
        jnp.zeros((6,), jnp.float32)
    ], axis=0)

    n_blocks = N_EDGES // BE
    return pl.pallas_call(
        _mlp_tc_body,
        grid=(n_blocks,),
        in_specs=[
            pl.BlockSpec((BE, LATENT), lambda i: (i, 0)),
            pl.BlockSpec((BE, 3), lambda i: (i, 0)),
            pl.BlockSpec((BE, 3), lambda i: (i, 0)),
            pl.BlockSpec((BE, 3), lambda i: (i, 0)),
            pl.BlockSpec((LATENT, 512), lambda i: (0, 0)),
            pl.BlockSpec((1, 512), lambda i: (0, 0)),
            pl.BlockSpec((512, 16), lambda i: (0, 0)),
            pl.BlockSpec((1, 16), lambda i: (0, 0)),
        ],
        out_specs=pl.BlockSpec((BE, 16), lambda i: (i, 0)),
        out_shape=jax.ShapeDtypeStruct((N_EDGES, 16), jnp.float32),
    )(interaction_latent, vector_a, vector_b, vector_c,
      w1, b1.reshape(1, 512), w2, b2.reshape(1, 16))


def kernel(edge_index, edge_attr, senders_pos, receivers_pos, vector_a,
           vector_b, vector_c, interaction_latent, w_nodes, node_type,
           i1_params, i2_params, fs_params, dx_params):
    vals16 = _decode_vals16(interaction_latent, vector_a, vector_b, vector_c,
                            i1_params, i2_params, fs_params, dx_params)

    senders = edge_index[0]
    receivers = edge_index[1]
    fij = vals16[:, 0:3]
    aij = vals16[:, 3:6]
    dxij = vals16[:, 6:9]
    lambda_ij = vals16[:, 9:10]

    is_global = node_type[:, -1] == -1
    is_virtual = (edge_attr == -1).reshape(-1)

    def remove_mean(t, mask, grp):
        mf = mask.astype(t.dtype)
        sums = jax.ops.segment_sum(t * mf[:, None], grp, num_segments=N_NODES)
        cnts = jax.ops.segment_sum(mf, grp, num_segments=N_NODES)
        mean = sums / jnp.maximum(cnts, 1.0)[:, None]
        return jnp.where(mask[:, None], t - mean[grp], t)

    mask_in = is_virtual & is_global[receivers]
    fij = remove_mean(fij, mask_in, receivers)
    aij = remove_mean(aij, mask_in, receivers)
    dxij = remove_mean(dxij, mask_in, receivers)

    mask_out = is_virtual & is_global[senders]
    fij = remove_mean(fij, mask_out, senders)
    aij = remove_mean(aij, mask_out, senders)
    dxij = remove_mean(dxij, mask_out, senders)

    w_s = w_nodes[senders]
    w_r = w_nodes[receivers]
    r0ij = (w_s * senders_pos + w_r * receivers_pos) / (w_s + w_r)
    lever_arm = receivers_pos - r0ij
    torque_contribution = jnp.cross(lever_arm, fij * lambda_ij, axis=1)
    tauij = aij - torque_contribution
    return (fij, tauij, dxij)


# speedup vs baseline: 4.5488x; 4.5488x over previous
import functools

import jax
import jax.numpy as jnp
from jax import lax
from jax.experimental import pallas as pl
from jax.experimental.pallas import tpu as pltpu
from jax.experimental.pallas import tpu_sc as plsc


N_NODES = 10000
N_EDGES = 320000
LATENT = 128
BE = 1600

NC = 2
NS = 16
C = 400
Q = 100
NQ = C // Q
NACC = 10240
SLICE = NACC // NS


def _mlp_tc_body(x_ref, va_ref, vb_ref, vc_ref, w1_ref, b1_ref, w2_ref, b2_ref,
                 out_ref):
    x = x_ref[...]
    h = jnp.maximum(
        lax.dot_general(x, w1_ref[...], (((1,), (0,)), ((), ())),
                        precision=lax.Precision.HIGHEST,
                        preferred_element_type=jnp.float32) + b1_ref[...],
        0.0)
    o = lax.dot_general(h, w2_ref[...], (((1,), (0,)), ((), ())),
                        precision=lax.Precision.HIGHEST,
                        preferred_element_type=jnp.float32) + b2_ref[...]
    va = va_ref[...]
    vb = vb_ref[...]
    vc = vc_ref[...]
    fij = o[:, 0:1] * va + o[:, 1:2] * vb + o[:, 2:3] * vc
    aij = o[:, 3:4] * va + o[:, 4:5] * vb + o[:, 5:6] * vc
    dxij = o[:, 6:7] * va + o[:, 7:8] * vb + o[:, 8:9] * vc
    out_ref[...] = jnp.concatenate(
        [fij, aij, dxij, o[:, 9:10], jnp.zeros((x.shape[0], 6), jnp.float32)],
        axis=1)


def _decode_vals16(interaction_latent, vector_a, vector_b, vector_c,
                   i1_params, i2_params, fs_params, dx_params):
    w1 = jnp.concatenate(
        [i1_params[0], i2_params[0], dx_params[0], fs_params[0]], axis=1)
    b1 = jnp.concatenate(
        [i1_params[1], i2_params[1], dx_params[1], fs_params[1]], axis=0)
    z = jnp.zeros((LATENT, 16), jnp.float32)
    w2 = jnp.concatenate([
        z.at[:, 0:3].set(i1_params[2]),
        z.at[:, 3:6].set(i2_params[2]),
        z.at[:, 6:9].set(dx_params[2]),
        z.at[:, 9:10].set(fs_params[2]),
    ], axis=0)
    b2 = jnp.concatenate([
        i1_params[3], i2_params[3], dx_params[3], fs_params[3],
        jnp.zeros((6,), jnp.float32)
    ], axis=0)

    n_blocks = N_EDGES // BE
    return pl.pallas_call(
        _mlp_tc_body,
        grid=(n_blocks,),
        in_specs=[
            pl.BlockSpec((BE, LATENT), lambda i: (i, 0)),
            pl.BlockSpec((BE, 3), lambda i: (i, 0)),
            pl.BlockSpec((BE, 3), lambda i: (i, 0)),
            pl.BlockSpec((BE, 3), lambda i: (i, 0)),
            pl.BlockSpec((LATENT, 512), lambda i: (0, 0)),
            pl.BlockSpec((1, 512), lambda i: (0, 0)),
            pl.BlockSpec((512, 16), lambda i: (0, 0)),
            pl.BlockSpec((1, 16), lambda i: (0, 0)),
        ],
        out_specs=pl.BlockSpec((BE, 16), lambda i: (i, 0)),
        out_shape=jax.ShapeDtypeStruct((N_EDGES, 16), jnp.float32),
    )(interaction_latent, vector_a, vector_b, vector_c,
      w1, b1.reshape(1, 512), w2, b2.reshape(1, 16))


def _lanes(i):
    return jax.lax.broadcasted_iota(jnp.int32, (16,), 0) + i * 16


def _full(c):
    return jnp.full((16,), c, jnp.int32)


def _mask_f32(eai, glob):
    hit = jnp.logical_and(eai == -1, glob == -1)
    return jnp.where(hit, 1.0, 0.0).astype(jnp.float32)


def _sc_body(vals16, snd1, rcv1, snd2, rcv2, eai1, ntl,
             wn, zer, sp_hbm, rp_hbm, fij_o, tau_o, dx_o,
             acc0, acc1, isg_t, w_t, rcv_v, snd_v, eai_v, rcv_i, snd_i,
             vals_t, rows_t, m0_t, m1_t, sp_t, rp_t, fo_t, to_t, do_t, ms_t):
    cid = lax.axis_index("c")
    sid = lax.axis_index("s")
    wid = cid * NS + sid

    pltpu.sync_copy(ntl, isg_t)
    pltpu.sync_copy(wn, w_t)
    pltpu.sync_copy(zer.at[pl.ds(0, C), :], rows_t)
    pltpu.sync_copy(zer.at[pl.ds(sid * SLICE, SLICE), :],
                    acc0.at[pl.ds(sid * SLICE, SLICE), :])
    pltpu.sync_copy(zer.at[pl.ds(sid * SLICE, SLICE), :],
                    acc1.at[pl.ds(sid * SLICE, SLICE), :])
    plsc.subcore_barrier()

    edges_per_sub = N_EDGES // NS
    chunks_acc = edges_per_sub // C
    edges_per_w = N_EDGES // (NC * NS)
    chunks_apply = edges_per_w // C

    def load_chunk_idx(base, need_snd):
        pltpu.sync_copy(rcv1.at[pl.ds(base, C)], rcv_v)
        pltpu.sync_copy(rcv2.at[pl.ds(base // Q, NQ), :], rcv_i)
        pltpu.sync_copy(eai1.at[pl.ds(base, C)], eai_v)
        if need_snd:
            pltpu.sync_copy(snd1.at[pl.ds(base, C)], snd_v)
            pltpu.sync_copy(snd2.at[pl.ds(base // Q, NQ), :], snd_i)

    def edge_masks(i, idx_v):
        g = plsc.load_gather(idx_v, [_lanes(i)])
        ea = plsc.load_gather(eai_v, [_lanes(i)])
        glob = plsc.load_gather(isg_t, [g])
        return g, _mask_f32(ea, glob)


    def p1_chunk(j, carry):
        base = sid * edges_per_sub + j * C
        load_chunk_idx(base, False)
        pltpu.sync_copy(vals16.at[pl.ds(base, C), :], vals_t)

        def p1_iter(i, carry2):
            rows = _lanes(i)
            _, m_in = edge_masks(i, rcv_v)
            for ch in range(9):
                v = plsc.load_gather(vals_t, [rows, _full(ch)])
                plsc.store_scatter(rows_t, [rows, _full(ch)], v * m_in)
            plsc.store_scatter(rows_t, [rows, _full(9)], m_in)
            return carry2

        lax.fori_loop(0, C // 16, p1_iter, 0)
        for q in range(NQ):
            pltpu.sync_copy(rows_t.at[pl.ds(q * Q, Q), :],
                            acc0.at[rcv_i.at[q]], add=True)
        return carry

    lax.fori_loop(0, chunks_acc, p1_chunk, 0)
    plsc.subcore_barrier()

    def make_means(acc):
        pltpu.sync_copy(acc.at[pl.ds(sid * SLICE, SLICE), :], ms_t)

        def mean_iter(i, carry):
            rows = _lanes(i)
            cnt = plsc.load_gather(ms_t, [rows, _full(9)])
            inv = 1.0 / jnp.maximum(cnt, 1.0)
            for ch in range(9):
                s = plsc.load_gather(ms_t, [rows, _full(ch)])
                plsc.store_scatter(ms_t, [rows, _full(ch)], s * inv)
            return carry

        lax.fori_loop(0, SLICE // 16, mean_iter, 0)
        pltpu.sync_copy(ms_t, acc.at[pl.ds(sid * SLICE, SLICE), :])

    make_means(acc0)
    plsc.subcore_barrier()


    def p2_chunk(j, carry):
        base = sid * edges_per_sub + j * C
        load_chunk_idx(base, True)
        pltpu.sync_copy(vals16.at[pl.ds(base, C), :], vals_t)
        for q in range(NQ):
            pltpu.sync_copy(acc0.at[rcv_i.at[q]],
                            m0_t.at[pl.ds(q * Q, Q), :])

        def p2_iter(i, carry2):
            rows = _lanes(i)
            _, m_in = edge_masks(i, rcv_v)
            _, m_out = edge_masks(i, snd_v)
            for ch in range(9):
                v = plsc.load_gather(vals_t, [rows, _full(ch)])
                m0 = plsc.load_gather(m0_t, [rows, _full(ch)])
                plsc.store_scatter(rows_t, [rows, _full(ch)],
                                   (v - m_in * m0) * m_out)
            plsc.store_scatter(rows_t, [rows, _full(9)], m_out)
            return carry2

        lax.fori_loop(0, C // 16, p2_iter, 0)
        for q in range(NQ):
            pltpu.sync_copy(rows_t.at[pl.ds(q * Q, Q), :],
                            acc1.at[snd_i.at[q]], add=True)
        return carry

    lax.fori_loop(0, chunks_acc, p2_chunk, 0)
    plsc.subcore_barrier()

    make_means(acc1)
    plsc.subcore_barrier()


    def p3_chunk(j, carry):
        base = wid * edges_per_w + j * C
        load_chunk_idx(base, True)
        pltpu.sync_copy(vals16.at[pl.ds(base, C), :], vals_t)
        pltpu.sync_copy(sp_hbm.at[pl.ds(base, C), :], sp_t)
        pltpu.sync_copy(rp_hbm.at[pl.ds(base, C), :], rp_t)
        for q in range(NQ):
            pltpu.sync_copy(acc0.at[rcv_i.at[q]],
                            m0_t.at[pl.ds(q * Q, Q), :])
            pltpu.sync_copy(acc1.at[snd_i.at[q]],
                            m1_t.at[pl.ds(q * Q, Q), :])

        def p3_iter(i, carry2):
            rows = _lanes(i)
            g_r, m_in = edge_masks(i, rcv_v)
            g_s, m_out = edge_masks(i, snd_v)
            vv = []
            for ch in range(9):
                v = plsc.load_gather(vals_t, [rows, _full(ch)])
                m0 = plsc.load_gather(m0_t, [rows, _full(ch)])
                m1 = plsc.load_gather(m1_t, [rows, _full(ch)])
                vv.append(v - m_in * m0 - m_out * m1)
            lam = plsc.load_gather(vals_t, [rows, _full(9)])
            w_s = plsc.load_gather(w_t, [g_s])
            w_r = plsc.load_gather(w_t, [g_r])
            inv = 1.0 / (w_s + w_r)
            ff = [vv[0] * lam, vv[1] * lam, vv[2] * lam]
            lever = []
            for k in range(3):
                sp = plsc.load_gather(sp_t, [rows, _full(k)])
                rp = plsc.load_gather(rp_t, [rows, _full(k)])
                r0 = (w_s * sp + w_r * rp) * inv
                lever.append(rp - r0)
            t0 = lever[1] * ff[2] - lever[2] * ff[1]
            t1 = lever[2] * ff[0] - lever[0] * ff[2]
            t2 = lever[0] * ff[1] - lever[1] * ff[0]
            taus = [vv[3] - t0, vv[4] - t1, vv[5] - t2]
            for k in range(3):
                plsc.store_scatter(fo_t, [rows, _full(k)], vv[k])
                plsc.store_scatter(to_t, [rows, _full(k)], taus[k])
                plsc.store_scatter(do_t, [rows, _full(k)], vv[6 + k])
            return carry2

        lax.fori_loop(0, C // 16, p3_iter, 0)
        pltpu.sync_copy(fo_t, fij_o.at[pl.ds(base, C), :])
        pltpu.sync_copy(to_t, tau_o.at[pl.ds(base, C), :])
        pltpu.sync_copy(do_t, dx_o.at[pl.ds(base, C), :])
        return carry

    lax.fori_loop(0, chunks_apply, p3_chunk, 0)


def _sc_decode(vals16, senders, receivers, edge_attr_flat, node_type_last,
               w_flat, senders_pos, receivers_pos):
    mesh = plsc.VectorSubcoreMesh(core_axis_name="c", subcore_axis_name="s",
                                  num_cores=NC, num_subcores=NS)

    out3 = jax.ShapeDtypeStruct((N_EDGES, 3), jnp.float32)
    f = pl.kernel(
        _sc_body,
        out_type=(out3, out3, out3),
        mesh=mesh,
        compiler_params=pltpu.CompilerParams(needs_layout_passes=False,
                                             use_tc_tiling_on_sc=False),
        scratch_types=[
            pltpu.VMEM_SHARED((NACC, 16), jnp.float32),
            pltpu.VMEM_SHARED((NACC, 16), jnp.float32),
            pltpu.VMEM((N_NODES,), jnp.int32),
            pltpu.VMEM((N_NODES,), jnp.float32),
            pltpu.VMEM((C,), jnp.int32),
            pltpu.VMEM((C,), jnp.int32),
            pltpu.VMEM((C,), jnp.int32),
            pltpu.VMEM((NQ, Q), jnp.int32),
            pltpu.VMEM((NQ, Q), jnp.int32),
            pltpu.VMEM((C, 16), jnp.float32),
            pltpu.VMEM((C, 16), jnp.float32),
            pltpu.VMEM((C, 16), jnp.float32),
            pltpu.VMEM((C, 16), jnp.float32),
            pltpu.VMEM((C, 3), jnp.float32),
            pltpu.VMEM((C, 3), jnp.float32),
            pltpu.VMEM((C, 3), jnp.float32),
            pltpu.VMEM((C, 3), jnp.float32),
            pltpu.VMEM((C, 3), jnp.float32),
            pltpu.VMEM((SLICE, 16), jnp.float32),
        ],
    )
    zer = jnp.zeros((NACC, 16), jnp.float32)
    snd2 = senders.reshape(N_EDGES // Q, Q)
    rcv2 = receivers.reshape(N_EDGES // Q, Q)
    return f(vals16, senders, receivers, snd2, rcv2,
             edge_attr_flat, node_type_last, w_flat, zer, senders_pos,
             receivers_pos)


def kernel(edge_index, edge_attr, senders_pos, receivers_pos, vector_a,
           vector_b, vector_c, interaction_latent, w_nodes, node_type,
           i1_params, i2_params, fs_params, dx_params):
    vals16 = _decode_vals16(interaction_latent, vector_a, vector_b, vector_c,
                            i1_params, i2_params, fs_params, dx_params)
    senders = edge_index[0].astype(jnp.int32)
    receivers = edge_index[1].astype(jnp.int32)
    fij, tau, dx = _sc_decode(
        vals16, senders, receivers,
        edge_attr.reshape(-1).astype(jnp.int32),
        node_type[:, -1].astype(jnp.int32),
        w_nodes.reshape(-1), senders_pos, receivers_pos)
    return (fij, tau, dx)


# speedup vs baseline: 6.3713x; 1.4006x over previous
import functools

import jax
import jax.numpy as jnp
from jax import lax
from jax.experimental import pallas as pl
from jax.experimental.pallas import tpu as pltpu
from jax.experimental.pallas import tpu_sc as plsc


N_NODES = 10000
N_EDGES = 320000
LATENT = 128
BE = 1600

NC = 2
NS = 16
C = 400
Q = 100
NQ = C // Q
NACC = 10240
SLICE = NACC // NS


def _mlp_tc_body(x_ref, va_ref, vb_ref, vc_ref, w1_ref, b1_ref, w2_ref, b2_ref,
                 out_ref):
    x = x_ref[...]
    h = jnp.maximum(
        lax.dot_general(x, w1_ref[...], (((1,), (0,)), ((), ())),
                        precision=lax.Precision.DEFAULT,
                        preferred_element_type=jnp.float32) + b1_ref[...],
        0.0)
    o = lax.dot_general(h, w2_ref[...], (((1,), (0,)), ((), ())),
                        precision=lax.Precision.DEFAULT,
                        preferred_element_type=jnp.float32) + b2_ref[...]
    va = va_ref[...]
    vb = vb_ref[...]
    vc = vc_ref[...]
    fij = o[:, 0:1] * va + o[:, 1:2] * vb + o[:, 2:3] * vc
    aij = o[:, 3:4] * va + o[:, 4:5] * vb + o[:, 5:6] * vc
    dxij = o[:, 6:7] * va + o[:, 7:8] * vb + o[:, 8:9] * vc
    out_ref[...] = jnp.concatenate(
        [fij, aij, dxij, o[:, 9:10], jnp.zeros((x.shape[0], 6), jnp.float32)],
        axis=1)


def _decode_vals16(interaction_latent, vector_a, vector_b, vector_c,
                   i1_params, i2_params, fs_params, dx_params):
    w1 = jnp.concatenate(
        [i1_params[0], i2_params[0], dx_params[0], fs_params[0]], axis=1)
    b1 = jnp.concatenate(
        [i1_params[1], i2_params[1], dx_params[1], fs_params[1]], axis=0)
    z = jnp.zeros((LATENT, 16), jnp.float32)
    w2 = jnp.concatenate([
        z.at[:, 0:3].set(i1_params[2]),
        z.at[:, 3:6].set(i2_params[2]),
        z.at[:, 6:9].set(dx_params[2]),
        z.at[:, 9:10].set(fs_params[2]),
    ], axis=0)
    b2 = jnp.concatenate([
        i1_params[3], i2_params[3], dx_params[3], fs_params[3],
        jnp.zeros((6,), jnp.float32)
    ], axis=0)

    n_blocks = N_EDGES // BE
    return pl.pallas_call(
        _mlp_tc_body,
        grid=(n_blocks,),
        in_specs=[
            pl.BlockSpec((BE, LATENT), lambda i: (i, 0)),
            pl.BlockSpec((BE, 3), lambda i: (i, 0)),
            pl.BlockSpec((BE, 3), lambda i: (i, 0)),
            pl.BlockSpec((BE, 3), lambda i: (i, 0)),
            pl.BlockSpec((LATENT, 512), lambda i: (0, 0)),
            pl.BlockSpec((1, 512), lambda i: (0, 0)),
            pl.BlockSpec((512, 16), lambda i: (0, 0)),
            pl.BlockSpec((1, 16), lambda i: (0, 0)),
        ],
        out_specs=pl.BlockSpec((BE, 16), lambda i: (i, 0)),
        out_shape=jax.ShapeDtypeStruct((N_EDGES, 16), jnp.float32),
    )(interaction_latent, vector_a, vector_b, vector_c,
      w1, b1.reshape(1, 512), w2, b2.reshape(1, 16))


def _lanes(i):
    return jax.lax.broadcasted_iota(jnp.int32, (16,), 0) + i * 16


def _full(c):
    return jnp.full((16,), c, jnp.int32)


def _mask_f32(eai, glob):
    hit = jnp.logical_and(eai == -1, glob == -1)
    return jnp.where(hit, 1.0, 0.0).astype(jnp.float32)


def _sc_body(vals16, snd1, rcv1, snd2, rcv2, eai1, ntl,
             wn, zer, sp_hbm, rp_hbm, fij_o, tau_o, dx_o,
             acc0, acc1, isg_t, w_t, rcv_v, snd_v, eai_v, rcv_i, snd_i,
             vals_t, rows_t, m0_t, m1_t, sp_t, rp_t, fo_t, to_t, do_t, ms_t):
    cid = lax.axis_index("c")
    sid = lax.axis_index("s")
    wid = cid * NS + sid

    pltpu.sync_copy(ntl, isg_t)
    pltpu.sync_copy(wn, w_t)
    pltpu.sync_copy(zer.at[pl.ds(0, C), :], rows_t)
    pltpu.sync_copy(zer.at[pl.ds(sid * SLICE, SLICE), :],
                    acc0.at[pl.ds(sid * SLICE, SLICE), :])
    pltpu.sync_copy(zer.at[pl.ds(sid * SLICE, SLICE), :],
                    acc1.at[pl.ds(sid * SLICE, SLICE), :])
    plsc.subcore_barrier()

    edges_per_sub = N_EDGES // NS
    chunks_acc = edges_per_sub // C
    edges_per_w = N_EDGES // (NC * NS)
    chunks_apply = edges_per_w // C

    def load_chunk_idx(base, need_snd):
        pltpu.sync_copy(rcv1.at[pl.ds(base, C)], rcv_v)
        pltpu.sync_copy(rcv2.at[pl.ds(base // Q, NQ), :], rcv_i)
        pltpu.sync_copy(eai1.at[pl.ds(base, C)], eai_v)
        if need_snd:
            pltpu.sync_copy(snd1.at[pl.ds(base, C)], snd_v)
            pltpu.sync_copy(snd2.at[pl.ds(base // Q, NQ), :], snd_i)

    def edge_masks(i, idx_v):
        g = plsc.load_gather(idx_v, [_lanes(i)])
        ea = plsc.load_gather(eai_v, [_lanes(i)])
        glob = plsc.load_gather(isg_t, [g])
        return g, _mask_f32(ea, glob)


    def p1_chunk(j, carry):
        base = sid * edges_per_sub + j * C
        load_chunk_idx(base, False)
        pltpu.sync_copy(vals16.at[pl.ds(base, C), :], vals_t)

        def p1_iter(i, carry2):
            rows = _lanes(i)
            _, m_in = edge_masks(i, rcv_v)
            for ch in range(9):
                v = plsc.load_gather(vals_t, [rows, _full(ch)])
                plsc.store_scatter(rows_t, [rows, _full(ch)], v * m_in)
            plsc.store_scatter(rows_t, [rows, _full(9)], m_in)
            return carry2

        lax.fori_loop(0, C // 16, p1_iter, 0)
        for q in range(NQ):
            pltpu.sync_copy(rows_t.at[pl.ds(q * Q, Q), :],
                            acc0.at[rcv_i.at[q]], add=True)
        return carry

    lax.fori_loop(0, chunks_acc, p1_chunk, 0)
    plsc.subcore_barrier()

    def make_means(acc):
        pltpu.sync_copy(acc.at[pl.ds(sid * SLICE, SLICE), :], ms_t)

        def mean_iter(i, carry):
            rows = _lanes(i)
            cnt = plsc.load_gather(ms_t, [rows, _full(9)])
            inv = 1.0 / jnp.maximum(cnt, 1.0)
            for ch in range(9):
                s = plsc.load_gather(ms_t, [rows, _full(ch)])
                plsc.store_scatter(ms_t, [rows, _full(ch)], s * inv)
            return carry

        lax.fori_loop(0, SLICE // 16, mean_iter, 0)
        pltpu.sync_copy(ms_t, acc.at[pl.ds(sid * SLICE, SLICE), :])

    make_means(acc0)
    plsc.subcore_barrier()


    def p2_chunk(j, carry):
        base = sid * edges_per_sub + j * C
        load_chunk_idx(base, True)
        pltpu.sync_copy(vals16.at[pl.ds(base, C), :], vals_t)
        for q in range(NQ):
            pltpu.sync_copy(acc0.at[rcv_i.at[q]],
                            m0_t.at[pl.ds(q * Q, Q), :])

        def p2_iter(i, carry2):
            rows = _lanes(i)
            _, m_in = edge_masks(i, rcv_v)
            _, m_out = edge_masks(i, snd_v)
            for ch in range(9):
                v = plsc.load_gather(vals_t, [rows, _full(ch)])
                m0 = plsc.load_gather(m0_t, [rows, _full(ch)])
                plsc.store_scatter(rows_t, [rows, _full(ch)],
                                   (v - m_in * m0) * m_out)
            plsc.store_scatter(rows_t, [rows, _full(9)], m_out)
            return carry2

        lax.fori_loop(0, C // 16, p2_iter, 0)
        for q in range(NQ):
            pltpu.sync_copy(rows_t.at[pl.ds(q * Q, Q), :],
                            acc1.at[snd_i.at[q]], add=True)
        return carry

    lax.fori_loop(0, chunks_acc, p2_chunk, 0)
    plsc.subcore_barrier()

    make_means(acc1)
    plsc.subcore_barrier()


    def p3_chunk(j, carry):
        base = wid * edges_per_w + j * C
        load_chunk_idx(base, True)
        pltpu.sync_copy(vals16.at[pl.ds(base, C), :], vals_t)
        pltpu.sync_copy(sp_hbm.at[pl.ds(base, C), :], sp_t)
        pltpu.sync_copy(rp_hbm.at[pl.ds(base, C), :], rp_t)
        for q in range(NQ):
            pltpu.sync_copy(acc0.at[rcv_i.at[q]],
                            m0_t.at[pl.ds(q * Q, Q), :])
            pltpu.sync_copy(acc1.at[snd_i.at[q]],
                            m1_t.at[pl.ds(q * Q, Q), :])

        def p3_iter(i, carry2):
            rows = _lanes(i)
            g_r, m_in = edge_masks(i, rcv_v)
            g_s, m_out = edge_masks(i, snd_v)
            vv = []
            for ch in range(9):
                v = plsc.load_gather(vals_t, [rows, _full(ch)])
                m0 = plsc.load_gather(m0_t, [rows, _full(ch)])
                m1 = plsc.load_gather(m1_t, [rows, _full(ch)])
                vv.append(v - m_in * m0 - m_out * m1)
            lam = plsc.load_gather(vals_t, [rows, _full(9)])
            w_s = plsc.load_gather(w_t, [g_s])
            w_r = plsc.load_gather(w_t, [g_r])
            inv = 1.0 / (w_s + w_r)
            ff = [vv[0] * lam, vv[1] * lam, vv[2] * lam]
            lever = []
            for k in range(3):
                sp = plsc.load_gather(sp_t, [rows, _full(k)])
                rp = plsc.load_gather(rp_t, [rows, _full(k)])
                r0 = (w_s * sp + w_r * rp) * inv
                lever.append(rp - r0)
            t0 = lever[1] * ff[2] - lever[2] * ff[1]
            t1 = lever[2] * ff[0] - lever[0] * ff[2]
            t2 = lever[0] * ff[1] - lever[1] * ff[0]
            taus = [vv[3] - t0, vv[4] - t1, vv[5] - t2]
            for k in range(3):
                plsc.store_scatter(fo_t, [rows, _full(k)], vv[k])
                plsc.store_scatter(to_t, [rows, _full(k)], taus[k])
                plsc.store_scatter(do_t, [rows, _full(k)], vv[6 + k])
            return carry2

        lax.fori_loop(0, C // 16, p3_iter, 0)
        pltpu.sync_copy(fo_t, fij_o.at[pl.ds(base, C), :])
        pltpu.sync_copy(to_t, tau_o.at[pl.ds(base, C), :])
        pltpu.sync_copy(do_t, dx_o.at[pl.ds(base, C), :])
        return carry

    lax.fori_loop(0, chunks_apply, p3_chunk, 0)


def _sc_decode(vals16, senders, receivers, edge_attr_flat, node_type_last,
               w_flat, senders_pos, receivers_pos):
    mesh = plsc.VectorSubcoreMesh(core_axis_name="c", subcore_axis_name="s",
                                  num_cores=NC, num_subcores=NS)

    out3 = jax.ShapeDtypeStruct((N_EDGES, 3), jnp.float32)
    f = pl.kernel(
        _sc_body,
        out_type=(out3, out3, out3),
        mesh=mesh,
        compiler_params=pltpu.CompilerParams(needs_layout_passes=False,
                                             use_tc_tiling_on_sc=False),
        scratch_types=[
            pltpu.VMEM_SHARED((NACC, 16), jnp.float32),
            pltpu.VMEM_SHARED((NACC, 16), jnp.float32),
            pltpu.VMEM((N_NODES,), jnp.int32),
            pltpu.VMEM((N_NODES,), jnp.float32),
            pltpu.VMEM((C,), jnp.int32),
            pltpu.VMEM((C,), jnp.int32),
            pltpu.VMEM((C,), jnp.int32),
            pltpu.VMEM((NQ, Q), jnp.int32),
            pltpu.VMEM((NQ, Q), jnp.int32),
            pltpu.VMEM((C, 16), jnp.float32),
            pltpu.VMEM((C, 16), jnp.float32),
            pltpu.VMEM((C, 16), jnp.float32),
            pltpu.VMEM((C, 16), jnp.float32),
            pltpu.VMEM((C, 3), jnp.float32),
            pltpu.VMEM((C, 3), jnp.float32),
            pltpu.VMEM((C, 3), jnp.float32),
            pltpu.VMEM((C, 3), jnp.float32),
            pltpu.VMEM((C, 3), jnp.float32),
            pltpu.VMEM((SLICE, 16), jnp.float32),
        ],
    )
    zer = jnp.zeros((NACC, 16), jnp.float32)
    snd2 = senders.reshape(N_EDGES // Q, Q)
    rcv2 = receivers.reshape(N_EDGES // Q, Q)
    return f(vals16, senders, receivers, snd2, rcv2,
             edge_attr_flat, node_type_last, w_flat, zer, senders_pos,
             receivers_pos)


def kernel(edge_index, edge_attr, senders_pos, receivers_pos, vector_a,
           vector_b, vector_c, interaction_latent, w_nodes, node_type,
           i1_params, i2_params, fs_params, dx_params):
    vals16 = _decode_vals16(interaction_latent, vector_a, vector_b, vector_c,
                            i1_params, i2_params, fs_params, dx_params)
    senders = edge_index[0].astype(jnp.int32)
    receivers = edge_index[1].astype(jnp.int32)
    fij, tau, dx = _sc_decode(
        vals16, senders, receivers,
        edge_attr.reshape(-1).astype(jnp.int32),
        node_type[:, -1].astype(jnp.int32),
        w_nodes.reshape(-1), senders_pos, receivers_pos)
    return (fij, tau, dx)


# speedup vs baseline: 6.5072x; 1.0213x over previous
import functools

import jax
import jax.numpy as jnp
from jax import lax
from jax.experimental import pallas as pl
from jax.experimental.pallas import tpu as pltpu
from jax.experimental.pallas import tpu_sc as plsc


N_NODES = 10000
N_EDGES = 320000
LATENT = 128
BE = 1600

NC = 2
NS = 16
C = 400
Q = 80
NQ = C // Q
NACC = 10240
SLICE = NACC // NS


def _mlp_tc_body(x_ref, va_ref, vb_ref, vc_ref, sp_ref, rp_ref, w1_ref,
                 b1_ref, w2_ref, b2_ref, out_ref):
    x = x_ref[...]
    h = jnp.maximum(
        lax.dot_general(x, w1_ref[...], (((1,), (0,)), ((), ())),
                        precision=lax.Precision.DEFAULT,
                        preferred_element_type=jnp.float32) + b1_ref[...],
        0.0)
    o = lax.dot_general(h, w2_ref[...], (((1,), (0,)), ((), ())),
                        precision=lax.Precision.DEFAULT,
                        preferred_element_type=jnp.float32) + b2_ref[...]
    va = va_ref[...]
    vb = vb_ref[...]
    vc = vc_ref[...]
    fij = o[:, 0:1] * va + o[:, 1:2] * vb + o[:, 2:3] * vc
    aij = o[:, 3:4] * va + o[:, 4:5] * vb + o[:, 5:6] * vc
    dxij = o[:, 6:7] * va + o[:, 7:8] * vb + o[:, 8:9] * vc
    out_ref[...] = jnp.concatenate(
        [fij, aij, dxij, o[:, 9:10], sp_ref[...], rp_ref[...]], axis=1)


def _decode_vals16(interaction_latent, vector_a, vector_b, vector_c,
                   senders_pos, receivers_pos,
                   i1_params, i2_params, fs_params, dx_params):
    w1 = jnp.concatenate(
        [i1_params[0], i2_params[0], dx_params[0], fs_params[0]], axis=1)
    b1 = jnp.concatenate(
        [i1_params[1], i2_params[1], dx_params[1], fs_params[1]], axis=0)
    z = jnp.zeros((LATENT, 16), jnp.float32)
    w2 = jnp.concatenate([
        z.at[:, 0:3].set(i1_params[2]),
        z.at[:, 3:6].set(i2_params[2]),
        z.at[:, 6:9].set(dx_params[2]),
        z.at[:, 9:10].set(fs_params[2]),
    ], axis=0)
    b2 = jnp.concatenate([
        i1_params[3], i2_params[3], dx_params[3], fs_params[3],
        jnp.zeros((6,), jnp.float32)
    ], axis=0)

    n_blocks = N_EDGES // BE
    return pl.pallas_call(
        _mlp_tc_body,
        grid=(n_blocks,),
        in_specs=[
            pl.BlockSpec((BE, LATENT), lambda i: (i, 0)),
            pl.BlockSpec((BE, 3), lambda i: (i, 0)),
            pl.BlockSpec((BE, 3), lambda i: (i, 0)),
            pl.BlockSpec((BE, 3), lambda i: (i, 0)),
            pl.BlockSpec((BE, 3), lambda i: (i, 0)),
            pl.BlockSpec((BE, 3), lambda i: (i, 0)),
            pl.BlockSpec((LATENT, 512), lambda i: (0, 0)),
            pl.BlockSpec((1, 512), lambda i: (0, 0)),
            pl.BlockSpec((512, 16), lambda i: (0, 0)),
            pl.BlockSpec((1, 16), lambda i: (0, 0)),
        ],
        out_specs=pl.BlockSpec((BE, 16), lambda i: (i, 0)),
        out_shape=jax.ShapeDtypeStruct((N_EDGES, 16), jnp.float32),
    )(interaction_latent, vector_a, vector_b, vector_c, senders_pos,
      receivers_pos, w1, b1.reshape(1, 512), w2, b2.reshape(1, 16))


def _lanes(i):
    return jax.lax.broadcasted_iota(jnp.int32, (16,), 0) + i * 16


def _full(c):
    return jnp.full((16,), c, jnp.int32)


def _mask_f32(eai, glob):
    hit = jnp.logical_and(eai == -1, glob == -1)
    return jnp.where(hit, 1.0, 0.0).astype(jnp.float32)


def _sc_body(vals16, snd1, rcv1, eai1, ntl,
             wn, zer, fij_o, tau_o, dx_o,
             acc0, acc1, isg_t, w_t, rcv_v, snd_v, eai_v, rcv_i, snd_i,
             vals_t, rows_t, m0_t, m1_t, fo_t, to_t, do_t, ms_t):
    cid = lax.axis_index("c")
    sid = lax.axis_index("s")
    wid = cid * NS + sid

    pltpu.sync_copy(ntl, isg_t)
    pltpu.sync_copy(wn, w_t)
    pltpu.sync_copy(zer.at[pl.ds(0, C), :], rows_t)
    pltpu.sync_copy(zer.at[pl.ds(sid * SLICE, SLICE), :],
                    acc0.at[pl.ds(sid * SLICE, SLICE), :])
    pltpu.sync_copy(zer.at[pl.ds(sid * SLICE, SLICE), :],
                    acc1.at[pl.ds(sid * SLICE, SLICE), :])
    plsc.subcore_barrier()

    edges_per_sub = N_EDGES // NS
    chunks_acc = edges_per_sub // C
    edges_per_w = N_EDGES // (NC * NS)
    chunks_apply = edges_per_w // C

    def load_chunk_idx(base, need_snd):
        pltpu.sync_copy(rcv1.at[pl.ds(base, C)], rcv_v)
        pltpu.sync_copy(eai1.at[pl.ds(base, C)], eai_v)
        for q in range(NQ):
            pltpu.sync_copy(rcv1.at[pl.ds(base + q * Q, Q)], rcv_i.at[q])
        if need_snd:
            pltpu.sync_copy(snd1.at[pl.ds(base, C)], snd_v)
            for q in range(NQ):
                pltpu.sync_copy(snd1.at[pl.ds(base + q * Q, Q)], snd_i.at[q])

    def edge_masks(i, idx_v):
        g = plsc.load_gather(idx_v, [_lanes(i)])
        ea = plsc.load_gather(eai_v, [_lanes(i)])
        glob = plsc.load_gather(isg_t, [g])
        return g, _mask_f32(ea, glob)


    def p1_chunk(j, carry):
        base = sid * edges_per_sub + j * C
        load_chunk_idx(base, False)
        pltpu.sync_copy(vals16.at[pl.ds(base, C), :], vals_t)

        def p1_iter(i, carry2):
            rows = _lanes(i)
            _, m_in = edge_masks(i, rcv_v)
            for ch in range(9):
                v = plsc.load_gather(vals_t, [rows, _full(ch)])
                plsc.store_scatter(rows_t, [rows, _full(ch)], v * m_in)
            plsc.store_scatter(rows_t, [rows, _full(9)], m_in)
            return carry2

        lax.fori_loop(0, C // 16, p1_iter, 0)
        for q in range(NQ):
            pltpu.sync_copy(rows_t.at[pl.ds(q * Q, Q), :],
                            acc0.at[rcv_i.at[q]], add=True)
        return carry

    lax.fori_loop(0, chunks_acc, p1_chunk, 0)
    plsc.subcore_barrier()

    def make_means(acc):
        pltpu.sync_copy(acc.at[pl.ds(sid * SLICE, SLICE), :], ms_t)

        def mean_iter(i, carry):
            rows = _lanes(i)
            cnt = plsc.load_gather(ms_t, [rows, _full(9)])
            inv = 1.0 / jnp.maximum(cnt, 1.0)
            for ch in range(9):
                s = plsc.load_gather(ms_t, [rows, _full(ch)])
                plsc.store_scatter(ms_t, [rows, _full(ch)], s * inv)
            return carry

        lax.fori_loop(0, SLICE // 16, mean_iter, 0)
        pltpu.sync_copy(ms_t, acc.at[pl.ds(sid * SLICE, SLICE), :])

    make_means(acc0)
    plsc.subcore_barrier()


    def p2_chunk(j, carry):
        base = sid * edges_per_sub + j * C
        load_chunk_idx(base, True)
        pltpu.sync_copy(vals16.at[pl.ds(base, C), :], vals_t)
        for q in range(NQ):
            pltpu.sync_copy(acc0.at[rcv_i.at[q]],
                            m0_t.at[pl.ds(q * Q, Q), :])

        def p2_iter(i, carry2):
            rows = _lanes(i)
            _, m_in = edge_masks(i, rcv_v)
            _, m_out = edge_masks(i, snd_v)
            for ch in range(9):
                v = plsc.load_gather(vals_t, [rows, _full(ch)])
                m0 = plsc.load_gather(m0_t, [rows, _full(ch)])
                plsc.store_scatter(rows_t, [rows, _full(ch)],
                                   (v - m_in * m0) * m_out)
            plsc.store_scatter(rows_t, [rows, _full(9)], m_out)
            return carry2

        lax.fori_loop(0, C // 16, p2_iter, 0)
        for q in range(NQ):
            pltpu.sync_copy(rows_t.at[pl.ds(q * Q, Q), :],
                            acc1.at[snd_i.at[q]], add=True)
        return carry

    lax.fori_loop(0, chunks_acc, p2_chunk, 0)
    plsc.subcore_barrier()

    make_means(acc1)
    plsc.subcore_barrier()


    def p3_chunk(j, carry):
        base = wid * edges_per_w + j * C
        load_chunk_idx(base, True)
        pltpu.sync_copy(vals16.at[pl.ds(base, C), :], vals_t)
        for q in range(NQ):
            pltpu.sync_copy(acc0.at[rcv_i.at[q]],
                            m0_t.at[pl.ds(q * Q, Q), :])
            pltpu.sync_copy(acc1.at[snd_i.at[q]],
                            m1_t.at[pl.ds(q * Q, Q), :])

        def p3_iter(i, carry2):
            rows = _lanes(i)
            g_r, m_in = edge_masks(i, rcv_v)
            g_s, m_out = edge_masks(i, snd_v)
            vv = []
            for ch in range(9):
                v = plsc.load_gather(vals_t, [rows, _full(ch)])
                m0 = plsc.load_gather(m0_t, [rows, _full(ch)])
                m1 = plsc.load_gather(m1_t, [rows, _full(ch)])
                vv.append(v - m_in * m0 - m_out * m1)
            lam = plsc.load_gather(vals_t, [rows, _full(9)])
            w_s = plsc.load_gather(w_t, [g_s])
            w_r = plsc.load_gather(w_t, [g_r])
            inv = 1.0 / (w_s + w_r)
            ff = [vv[0] * lam, vv[1] * lam, vv[2] * lam]
            lever = []
            for k in range(3):
                sp = plsc.load_gather(vals_t, [rows, _full(10 + k)])
                rp = plsc.load_gather(vals_t, [rows, _full(13 + k)])
                r0 = (w_s * sp + w_r * rp) * inv
                lever.append(rp - r0)
            t0 = lever[1] * ff[2] - lever[2] * ff[1]
            t1 = lever[2] * ff[0] - lever[0] * ff[2]
            t2 = lever[0] * ff[1] - lever[1] * ff[0]
            taus = [vv[3] - t0, vv[4] - t1, vv[5] - t2]
            for k in range(3):
                plsc.store_scatter(fo_t, [rows, _full(k)], vv[k])
                plsc.store_scatter(to_t, [rows, _full(k)], taus[k])
                plsc.store_scatter(do_t, [rows, _full(k)], vv[6 + k])
            return carry2

        lax.fori_loop(0, C // 16, p3_iter, 0)
        pltpu.sync_copy(fo_t, fij_o.at[pl.ds(base, C), :])
        pltpu.sync_copy(to_t, tau_o.at[pl.ds(base, C), :])
        pltpu.sync_copy(do_t, dx_o.at[pl.ds(base, C), :])
        return carry

    lax.fori_loop(0, chunks_apply, p3_chunk, 0)


def _sc_decode(vals16, senders, receivers, edge_attr_flat, node_type_last,
               w_flat):
    mesh = plsc.VectorSubcoreMesh(core_axis_name="c", subcore_axis_name="s",
                                  num_cores=NC, num_subcores=NS)

    out3 = jax.ShapeDtypeStruct((N_EDGES, 3), jnp.float32)
    f = pl.kernel(
        _sc_body,
        out_type=(out3, out3, out3),
        mesh=mesh,
        compiler_params=pltpu.CompilerParams(needs_layout_passes=False,
                                             use_tc_tiling_on_sc=False),
        scratch_types=[
            pltpu.VMEM_SHARED((NACC, 16), jnp.float32),
            pltpu.VMEM_SHARED((NACC, 16), jnp.float32),
            pltpu.VMEM((N_NODES,), jnp.int32),
            pltpu.VMEM((N_NODES,), jnp.float32),
            pltpu.VMEM((C,), jnp.int32),
            pltpu.VMEM((C,), jnp.int32),
            pltpu.VMEM((C,), jnp.int32),
            pltpu.VMEM((NQ, Q), jnp.int32),
            pltpu.VMEM((NQ, Q), jnp.int32),
            pltpu.VMEM((C, 16), jnp.float32),
            pltpu.VMEM((C, 16), jnp.float32),
            pltpu.VMEM((C, 16), jnp.float32),
            pltpu.VMEM((C, 16), jnp.float32),
            pltpu.VMEM((C, 3), jnp.float32),
            pltpu.VMEM((C, 3), jnp.float32),
            pltpu.VMEM((C, 3), jnp.float32),
            pltpu.VMEM((SLICE, 16), jnp.float32),
        ],
    )
    zer = jnp.zeros((NACC, 16), jnp.float32)
    return f(vals16, senders, receivers,
             edge_attr_flat, node_type_last, w_flat, zer)


def kernel(edge_index, edge_attr, senders_pos, receivers_pos, vector_a,
           vector_b, vector_c, interaction_latent, w_nodes, node_type,
           i1_params, i2_params, fs_params, dx_params):
    vals16 = _decode_vals16(interaction_latent, vector_a, vector_b, vector_c,
                            senders_pos, receivers_pos,
                            i1_params, i2_params, fs_params, dx_params)
    senders = edge_index[0].astype(jnp.int32)
    receivers = edge_index[1].astype(jnp.int32)
    fij, tau, dx = _sc_decode(
        vals16, senders, receivers,
        edge_attr.reshape(-1).astype(jnp.int32),
        node_type[:, -1].astype(jnp.int32),
        w_nodes.reshape(-1))
    return (fij, tau, dx)


# speedup vs baseline: 14.4271x; 2.2171x over previous
import functools

import jax
import jax.numpy as jnp
from jax import lax
from jax.experimental import pallas as pl
from jax.experimental.pallas import tpu as pltpu
from jax.experimental.pallas import tpu_sc as plsc


N_NODES = 10000
N_EDGES = 320000
LATENT = 128
BE = 2560

NC = 2
NS = 16
C = 400
Q = 80
NQ = C // Q
NACC = 10240
SLICE = NACC // NS


def _mlp_tc_body(x_ref, va_ref, vb_ref, vc_ref, sp_ref, rp_ref, w1_ref,
                 b1_ref, w2t_ref, b2t_ref, out_ref):
    x = x_ref[...]
    h = jnp.maximum(
        lax.dot_general(x, w1_ref[...], (((1,), (0,)), ((), ())),
                        preferred_element_type=jnp.float32) + b1_ref[...],
        0.0)
    ot = lax.dot_general(w2t_ref[...], h, (((1,), (1,)), ((), ())),
                         preferred_element_type=jnp.float32) + b2t_ref[...]
    va = va_ref[...]
    vb = vb_ref[...]
    vc = vc_ref[...]
    fij = ot[0:1] * va + ot[1:2] * vb + ot[2:3] * vc
    aij = ot[3:4] * va + ot[4:5] * vb + ot[5:6] * vc
    dxij = ot[6:7] * va + ot[7:8] * vb + ot[8:9] * vc
    out_ref[...] = jnp.concatenate(
        [fij, aij, dxij, ot[9:10], sp_ref[...], rp_ref[...]], axis=0)


def _decode_vals16(interaction_latent, vector_a, vector_b, vector_c,
                   senders_pos, receivers_pos,
                   i1_params, i2_params, fs_params, dx_params):
    w1 = jnp.concatenate(
        [i1_params[0], i2_params[0], dx_params[0], fs_params[0]], axis=1)
    b1 = jnp.concatenate(
        [i1_params[1], i2_params[1], dx_params[1], fs_params[1]], axis=0)
    z = jnp.zeros((LATENT, 16), jnp.float32)
    w2 = jnp.concatenate([
        z.at[:, 0:3].set(i1_params[2]),
        z.at[:, 3:6].set(i2_params[2]),
        z.at[:, 6:9].set(dx_params[2]),
        z.at[:, 9:10].set(fs_params[2]),
    ], axis=0)
    b2 = jnp.concatenate([
        i1_params[3], i2_params[3], dx_params[3], fs_params[3],
        jnp.zeros((6,), jnp.float32)
    ], axis=0)

    n_blocks = N_EDGES // BE
    return pl.pallas_call(
        _mlp_tc_body,
        grid=(n_blocks,),
        in_specs=[
            pl.BlockSpec((BE, LATENT), lambda i: (i, 0)),
            pl.BlockSpec((3, BE), lambda i: (0, i)),
            pl.BlockSpec((3, BE), lambda i: (0, i)),
            pl.BlockSpec((3, BE), lambda i: (0, i)),
            pl.BlockSpec((3, BE), lambda i: (0, i)),
            pl.BlockSpec((3, BE), lambda i: (0, i)),
            pl.BlockSpec((LATENT, 512), lambda i: (0, 0)),
            pl.BlockSpec((1, 512), lambda i: (0, 0)),
            pl.BlockSpec((16, 512), lambda i: (0, 0)),
            pl.BlockSpec((16, 1), lambda i: (0, 0)),
        ],
        out_specs=pl.BlockSpec((16, BE), lambda i: (0, i)),
        out_shape=jax.ShapeDtypeStruct((16, N_EDGES), jnp.float32),
    )(interaction_latent, vector_a.T, vector_b.T, vector_c.T, senders_pos.T,
      receivers_pos.T, w1, b1.reshape(1, 512), w2.T, b2.reshape(16, 1))


def _lanes(i):
    return jax.lax.broadcasted_iota(jnp.int32, (16,), 0) + i * 16


def _full(c):
    return jnp.full((16,), c, jnp.int32)


def _mask_f32(eai, glob):
    hit = jnp.logical_and(eai == -1, glob == -1)
    return jnp.where(hit, 1.0, 0.0).astype(jnp.float32)


def _sc_body(vals16, snd1, rcv1, eai1, ntl, wn, zer, fij_o, tau_o, dx_o,
             acc0, acc1, isg_t, w_t, rcv_v, snd_v, eai_v, rcv_i, snd_i,
             vals_t, rows_t, m0_t, m1_t, fo_t, to_t, do_t, ms_t):
    cid = lax.axis_index("c")
    sid = lax.axis_index("s")
    wid = cid * NS + sid

    pltpu.sync_copy(ntl, isg_t)
    pltpu.sync_copy(wn, w_t)
    pltpu.sync_copy(zer.at[pl.ds(0, C), :], rows_t)
    pltpu.sync_copy(zer.at[pl.ds(sid * SLICE, SLICE), :],
                    acc0.at[pl.ds(sid * SLICE, SLICE), :])
    pltpu.sync_copy(zer.at[pl.ds(sid * SLICE, SLICE), :],
                    acc1.at[pl.ds(sid * SLICE, SLICE), :])
    plsc.subcore_barrier()

    edges_per_sub = N_EDGES // NS
    chunks_acc = edges_per_sub // C
    edges_per_w = N_EDGES // (NC * NS)
    chunks_apply = edges_per_w // C

    def load_chunk_idx(base, need_snd):
        pltpu.sync_copy(rcv1.at[pl.ds(base, C)], rcv_v)
        pltpu.sync_copy(eai1.at[pl.ds(base, C)], eai_v)
        for q in range(NQ):
            pltpu.sync_copy(rcv1.at[pl.ds(base + q * Q, Q)], rcv_i.at[q])
        if need_snd:
            pltpu.sync_copy(snd1.at[pl.ds(base, C)], snd_v)
            for q in range(NQ):
                pltpu.sync_copy(snd1.at[pl.ds(base + q * Q, Q)], snd_i.at[q])

    def load_vals(base, nch):
        pltpu.sync_copy(vals16.at[pl.ds(0, nch), pl.ds(base, C)],
                        vals_t.at[pl.ds(0, nch), :])

    def edge_masks(i, idx_v):
        g = plsc.load_gather(idx_v, [_lanes(i)])
        ea = plsc.load_gather(eai_v, [_lanes(i)])
        glob = plsc.load_gather(isg_t, [g])
        return g, _mask_f32(ea, glob)

    def vrow(ch, i):
        return plsc.load_gather(vals_t, [_full(ch), _lanes(i)])

    def p1_chunk(j, carry):
        base = sid * edges_per_sub + j * C
        load_chunk_idx(base, False)
        load_vals(base, 9)

        def p1_iter(i, carry2):
            rows = _lanes(i)
            _, m_in = edge_masks(i, rcv_v)
            for ch in range(9):
                plsc.store_scatter(rows_t, [rows, _full(ch)],
                                   vrow(ch, i) * m_in)
            plsc.store_scatter(rows_t, [rows, _full(9)], m_in)
            return carry2

        lax.fori_loop(0, C // 16, p1_iter, 0)
        for q in range(NQ):
            pltpu.sync_copy(rows_t.at[pl.ds(q * Q, Q), :],
                            acc0.at[rcv_i.at[q]], add=True)
        return carry

    lax.fori_loop(0, chunks_acc, p1_chunk, 0)
    plsc.subcore_barrier()

    def make_means(acc):
        pltpu.sync_copy(acc.at[pl.ds(sid * SLICE, SLICE), :], ms_t)

        def mean_iter(i, carry):
            rows = _lanes(i)
            cnt = plsc.load_gather(ms_t, [rows, _full(9)])
            inv = 1.0 / jnp.maximum(cnt, 1.0)
            for ch in range(9):
                s = plsc.load_gather(ms_t, [rows, _full(ch)])
                plsc.store_scatter(ms_t, [rows, _full(ch)], s * inv)
            return carry

        lax.fori_loop(0, SLICE // 16, mean_iter, 0)
        pltpu.sync_copy(ms_t, acc.at[pl.ds(sid * SLICE, SLICE), :])

    make_means(acc0)
    plsc.subcore_barrier()

    def p2_chunk(j, carry):
        base = sid * edges_per_sub + j * C
        load_chunk_idx(base, True)
        load_vals(base, 9)
        for q in range(NQ):
            pltpu.sync_copy(acc0.at[rcv_i.at[q]],
                            m0_t.at[pl.ds(q * Q, Q), :])

        def p2_iter(i, carry2):
            rows = _lanes(i)
            _, m_in = edge_masks(i, rcv_v)
            _, m_out = edge_masks(i, snd_v)
            for ch in range(9):
                m0 = plsc.load_gather(m0_t, [rows, _full(ch)])
                plsc.store_scatter(rows_t, [rows, _full(ch)],
                                   (vrow(ch, i) - m_in * m0) * m_out)
            plsc.store_scatter(rows_t, [rows, _full(9)], m_out)
            return carry2

        lax.fori_loop(0, C // 16, p2_iter, 0)
        for q in range(NQ):
            pltpu.sync_copy(rows_t.at[pl.ds(q * Q, Q), :],
                            acc1.at[snd_i.at[q]], add=True)
        return carry

    lax.fori_loop(0, chunks_acc, p2_chunk, 0)
    plsc.subcore_barrier()

    make_means(acc1)
    plsc.subcore_barrier()

    def p3_chunk(j, carry):
        base = wid * edges_per_w + j * C
        load_chunk_idx(base, True)
        load_vals(base, 16)
        for q in range(NQ):
            pltpu.sync_copy(acc0.at[rcv_i.at[q]],
                            m0_t.at[pl.ds(q * Q, Q), :])
            pltpu.sync_copy(acc1.at[snd_i.at[q]],
                            m1_t.at[pl.ds(q * Q, Q), :])

        def p3_iter(i, carry2):
            rows = _lanes(i)
            g_r, m_in = edge_masks(i, rcv_v)
            g_s, m_out = edge_masks(i, snd_v)
            vv = []
            for ch in range(9):
                m0 = plsc.load_gather(m0_t, [rows, _full(ch)])
                m1 = plsc.load_gather(m1_t, [rows, _full(ch)])
                vv.append(vrow(ch, i) - m_in * m0 - m_out * m1)
            lam = vrow(9, i)
            w_s = plsc.load_gather(w_t, [g_s])
            w_r = plsc.load_gather(w_t, [g_r])
            inv = 1.0 / (w_s + w_r)
            ff = [vv[0] * lam, vv[1] * lam, vv[2] * lam]
            lever = []
            for k in range(3):
                sp = vrow(10 + k, i)
                rp = vrow(13 + k, i)
                r0 = (w_s * sp + w_r * rp) * inv
                lever.append(rp - r0)
            t0 = lever[1] * ff[2] - lever[2] * ff[1]
            t1 = lever[2] * ff[0] - lever[0] * ff[2]
            t2 = lever[0] * ff[1] - lever[1] * ff[0]
            taus = [vv[3] - t0, vv[4] - t1, vv[5] - t2]
            for k in range(3):
                plsc.store_scatter(fo_t, [_full(k), rows], vv[k])
                plsc.store_scatter(to_t, [_full(k), rows], taus[k])
                plsc.store_scatter(do_t, [_full(k), rows], vv[6 + k])
            return carry2

        lax.fori_loop(0, C // 16, p3_iter, 0)
        pltpu.sync_copy(fo_t, fij_o.at[:, pl.ds(base, C)])
        pltpu.sync_copy(to_t, tau_o.at[:, pl.ds(base, C)])
        pltpu.sync_copy(do_t, dx_o.at[:, pl.ds(base, C)])
        return carry

    lax.fori_loop(0, chunks_apply, p3_chunk, 0)


def _sc_decode(vals16, senders, receivers, edge_attr_flat, node_type_last,
               w_flat):
    mesh = plsc.VectorSubcoreMesh(core_axis_name="c", subcore_axis_name="s",
                                  num_cores=NC, num_subcores=NS)

    out3 = jax.ShapeDtypeStruct((3, N_EDGES), jnp.float32)
    f = pl.kernel(
        _sc_body,
        out_type=(out3, out3, out3),
        mesh=mesh,
        compiler_params=pltpu.CompilerParams(needs_layout_passes=False,
                                             use_tc_tiling_on_sc=False),
        scratch_types=[
            pltpu.VMEM_SHARED((NACC, 16), jnp.float32),
            pltpu.VMEM_SHARED((NACC, 16), jnp.float32),
            pltpu.VMEM((N_NODES,), jnp.int32),
            pltpu.VMEM((N_NODES,), jnp.float32),
            pltpu.VMEM((C,), jnp.int32),
            pltpu.VMEM((C,), jnp.int32),
            pltpu.VMEM((C,), jnp.int32),
            pltpu.VMEM((NQ, Q), jnp.int32),
            pltpu.VMEM((NQ, Q), jnp.int32),
            pltpu.VMEM((16, C), jnp.float32),
            pltpu.VMEM((C, 16), jnp.float32),
            pltpu.VMEM((C, 16), jnp.float32),
            pltpu.VMEM((C, 16), jnp.float32),
            pltpu.VMEM((3, C), jnp.float32),
            pltpu.VMEM((3, C), jnp.float32),
            pltpu.VMEM((3, C), jnp.float32),
            pltpu.VMEM((SLICE, 16), jnp.float32),
        ],
    )
    zer = jnp.zeros((NACC, 16), jnp.float32)
    return f(vals16, senders, receivers,
             edge_attr_flat, node_type_last, w_flat, zer)


def kernel(edge_index, edge_attr, senders_pos, receivers_pos, vector_a,
           vector_b, vector_c, interaction_latent, w_nodes, node_type,
           i1_params, i2_params, fs_params, dx_params):
    vals16 = _decode_vals16(interaction_latent, vector_a, vector_b, vector_c,
                            senders_pos, receivers_pos,
                            i1_params, i2_params, fs_params, dx_params)
    senders = edge_index[0].astype(jnp.int32)
    receivers = edge_index[1].astype(jnp.int32)
    fij_t, tau_t, dx_t = _sc_decode(
        vals16, senders, receivers,
        edge_attr.reshape(-1).astype(jnp.int32),
        node_type[:, -1].astype(jnp.int32),
        w_nodes.reshape(-1))
    return (fij_t.T, tau_t.T, dx_t.T)


# speedup vs baseline: 27.3623x; 1.8966x over previous
import functools

import jax
import jax.numpy as jnp
from jax import lax
from jax.experimental import pallas as pl
from jax.experimental.pallas import tpu as pltpu
from jax.experimental.pallas import tpu_sc as plsc


N_NODES = 10000
N_EDGES = 320000
LATENT = 128
BE = 2560

NC = 2
NS = 16
C = 400
Q = 80
NQ = C // Q
NACC = 10240
SLICE = NACC // NS


def _mlp_tc_body(x_ref, va_ref, vb_ref, vc_ref, sp_ref, rp_ref, w1_ref,
                 b1_ref, w2t_ref, b2t_ref, out_ref):
    x = x_ref[...]
    h = jnp.maximum(
        lax.dot_general(x, w1_ref[...], (((1,), (0,)), ((), ())),
                        preferred_element_type=jnp.float32) + b1_ref[...],
        0.0)
    ot = lax.dot_general(w2t_ref[...], h, (((1,), (1,)), ((), ())),
                         preferred_element_type=jnp.float32) + b2t_ref[...]
    va = va_ref[...]
    vb = vb_ref[...]
    vc = vc_ref[...]
    fij = ot[0:1] * va + ot[1:2] * vb + ot[2:3] * vc
    aij = ot[3:4] * va + ot[4:5] * vb + ot[5:6] * vc
    dxij = ot[6:7] * va + ot[7:8] * vb + ot[8:9] * vc
    out_ref[...] = jnp.concatenate(
        [fij, aij, dxij, ot[9:10], sp_ref[...], rp_ref[...]], axis=0)


def _decode_vals16(interaction_latent, vector_a, vector_b, vector_c,
                   senders_pos, receivers_pos,
                   i1_params, i2_params, fs_params, dx_params):
    w1 = jnp.concatenate(
        [i1_params[0], i2_params[0], dx_params[0], fs_params[0]], axis=1)
    b1 = jnp.concatenate(
        [i1_params[1], i2_params[1], dx_params[1], fs_params[1]], axis=0)
    z = jnp.zeros((LATENT, 16), jnp.float32)
    w2 = jnp.concatenate([
        z.at[:, 0:3].set(i1_params[2]),
        z.at[:, 3:6].set(i2_params[2]),
        z.at[:, 6:9].set(dx_params[2]),
        z.at[:, 9:10].set(fs_params[2]),
    ], axis=0)
    b2 = jnp.concatenate([
        i1_params[3], i2_params[3], dx_params[3], fs_params[3],
        jnp.zeros((6,), jnp.float32)
    ], axis=0)

    n_blocks = N_EDGES // BE
    return pl.pallas_call(
        _mlp_tc_body,
        grid=(n_blocks,),
        in_specs=[
            pl.BlockSpec((BE, LATENT), lambda i: (i, 0)),
            pl.BlockSpec((3, BE), lambda i: (0, i)),
            pl.BlockSpec((3, BE), lambda i: (0, i)),
            pl.BlockSpec((3, BE), lambda i: (0, i)),
            pl.BlockSpec((3, BE), lambda i: (0, i)),
            pl.BlockSpec((3, BE), lambda i: (0, i)),
            pl.BlockSpec((LATENT, 512), lambda i: (0, 0)),
            pl.BlockSpec((1, 512), lambda i: (0, 0)),
            pl.BlockSpec((16, 512), lambda i: (0, 0)),
            pl.BlockSpec((16, 1), lambda i: (0, 0)),
        ],
        out_specs=pl.BlockSpec((16, BE), lambda i: (0, i)),
        out_shape=jax.ShapeDtypeStruct((16, N_EDGES), jnp.float32),
    )(interaction_latent, vector_a.T, vector_b.T, vector_c.T, senders_pos.T,
      receivers_pos.T, w1, b1.reshape(1, 512), w2.T, b2.reshape(16, 1))


def _lanes(i):
    return jax.lax.broadcasted_iota(jnp.int32, (16,), 0) + i * 16


def _full(c):
    return jnp.full((16,), c, jnp.int32)


def _mask_f32(eai, glob):
    hit = jnp.logical_and(eai == -1, glob == -1)
    return jnp.where(hit, 1.0, 0.0).astype(jnp.float32)


def _sc_body(vals16, snd1, rcv1, eai1, ntl, wn, zer, fij_o, tau_o, dx_o,
             acc0, acc1, isg_t, w_t, rcv_v, snd_v, eai_v, rcv_i, snd_i,
             vals_t, rows_t, m0_t, m1_t, fo_t, to_t, do_t, ms_t, sem):
    cid = lax.axis_index("c")
    sid = lax.axis_index("s")
    wid = cid * NS + sid

    pltpu.sync_copy(ntl, isg_t)
    pltpu.sync_copy(wn, w_t)
    pltpu.sync_copy(zer.at[pl.ds(0, C), :], rows_t)
    pltpu.sync_copy(zer.at[pl.ds(sid * SLICE, SLICE), :],
                    acc0.at[pl.ds(sid * SLICE, SLICE), :])
    pltpu.sync_copy(zer.at[pl.ds(sid * SLICE, SLICE), :],
                    acc1.at[pl.ds(sid * SLICE, SLICE), :])
    plsc.subcore_barrier()

    edges_per_sub = N_EDGES // NS
    chunks_acc = edges_per_sub // C
    edges_per_w = N_EDGES // (NC * NS)
    chunks_apply = edges_per_w // C

    def drain(ds):
        for d in ds:
            d.wait()

    def load_chunk(base, need_snd, nch):
        ds = [pltpu.async_copy(rcv1.at[pl.ds(base, C)], rcv_v, sem),
              pltpu.async_copy(eai1.at[pl.ds(base, C)], eai_v, sem),
              pltpu.async_copy(vals16.at[pl.ds(0, nch), pl.ds(base, C)],
                               vals_t.at[pl.ds(0, nch), :], sem)]
        for q in range(NQ):
            ds.append(pltpu.async_copy(rcv1.at[pl.ds(base + q * Q, Q)],
                                       rcv_i.at[q], sem))
        if need_snd:
            ds.append(pltpu.async_copy(snd1.at[pl.ds(base, C)], snd_v, sem))
            for q in range(NQ):
                ds.append(pltpu.async_copy(snd1.at[pl.ds(base + q * Q, Q)],
                                           snd_i.at[q], sem))
        drain(ds)

    def edge_masks(i, idx_v):
        g = plsc.load_gather(idx_v, [_lanes(i)])
        ea = plsc.load_gather(eai_v, [_lanes(i)])
        glob = plsc.load_gather(isg_t, [g])
        return g, _mask_f32(ea, glob)

    def vrow(ch, i):
        return plsc.load_gather(vals_t, [_full(ch), _lanes(i)])

    def p1_chunk(j, carry):
        base = sid * edges_per_sub + j * C
        load_chunk(base, False, 9)

        def p1_iter(i, carry2):
            rows = _lanes(i)
            _, m_in = edge_masks(i, rcv_v)
            for ch in range(9):
                plsc.store_scatter(rows_t, [rows, _full(ch)],
                                   vrow(ch, i) * m_in)
            plsc.store_scatter(rows_t, [rows, _full(9)], m_in)
            return carry2

        lax.fori_loop(0, C // 16, p1_iter, 0)
        drain([pltpu.async_copy(rows_t.at[pl.ds(q * Q, Q), :],
                                acc0.at[rcv_i.at[q]], sem, add=True)
               for q in range(NQ)])
        return carry

    lax.fori_loop(0, chunks_acc, p1_chunk, 0)
    plsc.subcore_barrier()

    def make_means(acc):
        pltpu.sync_copy(acc.at[pl.ds(sid * SLICE, SLICE), :], ms_t)

        def mean_iter(i, carry):
            rows = _lanes(i)
            cnt = plsc.load_gather(ms_t, [rows, _full(9)])
            inv = 1.0 / jnp.maximum(cnt, 1.0)
            for ch in range(9):
                s = plsc.load_gather(ms_t, [rows, _full(ch)])
                plsc.store_scatter(ms_t, [rows, _full(ch)], s * inv)
            return carry

        lax.fori_loop(0, SLICE // 16, mean_iter, 0)
        pltpu.sync_copy(ms_t, acc.at[pl.ds(sid * SLICE, SLICE), :])

    make_means(acc0)
    plsc.subcore_barrier()

    def p2_chunk(j, carry):
        base = sid * edges_per_sub + j * C
        load_chunk(base, True, 9)
        drain([pltpu.async_copy(acc0.at[rcv_i.at[q]],
                                m0_t.at[pl.ds(q * Q, Q), :], sem)
               for q in range(NQ)])

        def p2_iter(i, carry2):
            rows = _lanes(i)
            _, m_in = edge_masks(i, rcv_v)
            _, m_out = edge_masks(i, snd_v)
            for ch in range(9):
                m0 = plsc.load_gather(m0_t, [rows, _full(ch)])
                plsc.store_scatter(rows_t, [rows, _full(ch)],
                                   (vrow(ch, i) - m_in * m0) * m_out)
            plsc.store_scatter(rows_t, [rows, _full(9)], m_out)
            return carry2

        lax.fori_loop(0, C // 16, p2_iter, 0)
        drain([pltpu.async_copy(rows_t.at[pl.ds(q * Q, Q), :],
                                acc1.at[snd_i.at[q]], sem, add=True)
               for q in range(NQ)])
        return carry

    lax.fori_loop(0, chunks_acc, p2_chunk, 0)
    plsc.subcore_barrier()

    make_means(acc1)
    plsc.subcore_barrier()

    def p3_chunk(j, carry):
        base = wid * edges_per_w + j * C
        load_chunk(base, True, 16)
        ds = []
        for q in range(NQ):
            ds.append(pltpu.async_copy(acc0.at[rcv_i.at[q]],
                                       m0_t.at[pl.ds(q * Q, Q), :], sem))
            ds.append(pltpu.async_copy(acc1.at[snd_i.at[q]],
                                       m1_t.at[pl.ds(q * Q, Q), :], sem))
        drain(ds)

        def p3_iter(i, carry2):
            rows = _lanes(i)
            g_r, m_in = edge_masks(i, rcv_v)
            g_s, m_out = edge_masks(i, snd_v)
            vv = []
            for ch in range(9):
                m0 = plsc.load_gather(m0_t, [rows, _full(ch)])
                m1 = plsc.load_gather(m1_t, [rows, _full(ch)])
                vv.append(vrow(ch, i) - m_in * m0 - m_out * m1)
            lam = vrow(9, i)
            w_s = plsc.load_gather(w_t, [g_s])
            w_r = plsc.load_gather(w_t, [g_r])
            inv = 1.0 / (w_s + w_r)
            ff = [vv[0] * lam, vv[1] * lam, vv[2] * lam]
            lever = []
            for k in range(3):
                sp = vrow(10 + k, i)
                rp = vrow(13 + k, i)
                r0 = (w_s * sp + w_r * rp) * inv
                lever.append(rp - r0)
            t0 = lever[1] * ff[2] - lever[2] * ff[1]
            t1 = lever[2] * ff[0] - lever[0] * ff[2]
            t2 = lever[0] * ff[1] - lever[1] * ff[0]
            taus = [vv[3] - t0, vv[4] - t1, vv[5] - t2]
            for k in range(3):
                plsc.store_scatter(fo_t, [_full(k), rows], vv[k])
                plsc.store_scatter(to_t, [_full(k), rows], taus[k])
                plsc.store_scatter(do_t, [_full(k), rows], vv[6 + k])
            return carry2

        lax.fori_loop(0, C // 16, p3_iter, 0)
        drain([pltpu.async_copy(fo_t, fij_o.at[:, pl.ds(base, C)], sem),
               pltpu.async_copy(to_t, tau_o.at[:, pl.ds(base, C)], sem),
               pltpu.async_copy(do_t, dx_o.at[:, pl.ds(base, C)], sem)])
        return carry

    lax.fori_loop(0, chunks_apply, p3_chunk, 0)


def _sc_decode(vals16, senders, receivers, edge_attr_flat, node_type_last,
               w_flat):
    mesh = plsc.VectorSubcoreMesh(core_axis_name="c", subcore_axis_name="s",
                                  num_cores=NC, num_subcores=NS)

    out3 = jax.ShapeDtypeStruct((3, N_EDGES), jnp.float32)
    f = pl.kernel(
        _sc_body,
        out_type=(out3, out3, out3),
        mesh=mesh,
        compiler_params=pltpu.CompilerParams(needs_layout_passes=False,
                                             use_tc_tiling_on_sc=False),
        scratch_types=[
            pltpu.VMEM_SHARED((NACC, 16), jnp.float32),
            pltpu.VMEM_SHARED((NACC, 16), jnp.float32),
            pltpu.VMEM((N_NODES,), jnp.int32),
            pltpu.VMEM((N_NODES,), jnp.float32),
            pltpu.VMEM((C,), jnp.int32),
            pltpu.VMEM((C,), jnp.int32),
            pltpu.VMEM((C,), jnp.int32),
            pltpu.VMEM((NQ, Q), jnp.int32),
            pltpu.VMEM((NQ, Q), jnp.int32),
            pltpu.VMEM((16, C), jnp.float32),
            pltpu.VMEM((C, 16), jnp.float32),
            pltpu.VMEM((C, 16), jnp.float32),
            pltpu.VMEM((C, 16), jnp.float32),
            pltpu.VMEM((3, C), jnp.float32),
            pltpu.VMEM((3, C), jnp.float32),
            pltpu.VMEM((3, C), jnp.float32),
            pltpu.VMEM((SLICE, 16), jnp.float32),
            pltpu.SemaphoreType.DMA,
        ],
    )
    zer = jnp.zeros((NACC, 16), jnp.float32)
    return f(vals16, senders, receivers,
             edge_attr_flat, node_type_last, w_flat, zer)


def kernel(edge_index, edge_attr, senders_pos, receivers_pos, vector_a,
           vector_b, vector_c, interaction_latent, w_nodes, node_type,
           i1_params, i2_params, fs_params, dx_params):
    vals16 = _decode_vals16(interaction_latent, vector_a, vector_b, vector_c,
                            senders_pos, receivers_pos,
                            i1_params, i2_params, fs_params, dx_params)
    senders = edge_index[0].astype(jnp.int32)
    receivers = edge_index[1].astype(jnp.int32)
    fij_t, tau_t, dx_t = _sc_decode(
        vals16, senders, receivers,
        edge_attr.reshape(-1).astype(jnp.int32),
        node_type[:, -1].astype(jnp.int32),
        w_nodes.reshape(-1))
    return (fij_t.T, tau_t.T, dx_t.T)
